# baseline (speedup 1.0000x reference)
import functools

import jax
import jax.numpy as jnp
from jax import lax
from jax.experimental import pallas as pl
from jax.experimental.pallas import tpu as pltpu
from jax.experimental.pallas import tpu_sc as plsc

_N = 10000
_D = 128
_NT = 16
_BLK = 400
_GRID = _N // _BLK

_NC = 2
_NS = 16
_NW = _NC * _NS
_CH = 64
_TOTCH = 5120
_NCH = _TOTCH // _NS
_NBUF = 5
_IRING = 10
_EPAD = _TOTCH * _CH
_NACC = _NS * 640


def _dot(a, b):
    return jnp.dot(a, b, preferred_element_type=jnp.float32)


def _dot_t(a, b):
    return lax.dot_general(a, b, (((1,), (1,)), ((), ())),
                           preferred_element_type=jnp.float32)


def _dot_c0(a, b):
    return lax.dot_general(a, b, (((0,), (0,)), ((), ())),
                           preferred_element_type=jnp.float32)


def _mask_linear_body(x_ref, mask_ref, w_ref, b_ref, o_ref):
    xm = x_ref[...] * (1.0 - mask_ref[...])
    o_ref[...] = _dot(xm, w_ref[...]) + b_ref[...]


def _gin_body(relu_out, h_ref, a0_ref, w1_ref, b1_ref, w2_ref, b2_ref,
              o_ref):
    a = h_ref[...] + a0_ref[...]
    t = jnp.maximum(_dot(a, w1_ref[...]) + b1_ref[...], 0.0)
    o = _dot(t, w2_ref[...]) + b2_ref[...]
    if relu_out:
        o = jnp.maximum(o, 0.0)
    o_ref[...] = o


def _latent_body(lat_ref, mx_ref, mp_ref, ctf_ref, wc_ref, bc_ref,
                 lx_ref, csum_ref, cnt_ref):
    i = pl.program_id(0)
    lat = lat_ref[...]
    lpos = _dot(lat, wc_ref[...]) + bc_ref[...]
    lpos = jnp.where(mp_ref[...] > 0.0, 0.0, lpos)
    lx_ref[...] = jnp.where(mx_ref[...] > 0.0, 0.0, lat)
    iota = lax.broadcasted_iota(jnp.int32, (1, _NT), 1).astype(jnp.float32)
    onehot = (ctf_ref[...] == iota).astype(jnp.float32)
    csum_p = _dot_c0(onehot, lpos)
    cnt_p = _dot_c0(onehot, jnp.ones((_BLK, 1), jnp.float32))

    @pl.when(i == 0)
    def _init():
        csum_ref[...] = jnp.zeros_like(csum_ref)
        cnt_ref[...] = jnp.zeros_like(cnt_ref)

    csum_ref[...] += csum_p
    cnt_ref[...] += cnt_p


def _decoder_body(lx_ref, a0_ref, x_ref, posp_ref, mx_ref, mp_ref, ctf_ref,
                  csum_ref, cnt_ref,
                  xw1_ref, xb1_ref, xw2_ref, xb2_ref,
                  pw1_ref, pb1_ref, pw2_ref, pb2_ref,
                  w2r_ref, b2r_ref,
                  slx_ref, slp_ref, smx_ref, smp_ref, sct_ref):
    i = pl.program_id(0)
    a = lx_ref[...] + a0_ref[...]
    t = jnp.maximum(_dot(a, xw1_ref[...]) + xb1_ref[...], 0.0)
    rx = _dot(t, xw2_ref[...]) + xb2_ref[...]
    t2 = jnp.maximum(_dot(a, pw1_ref[...]) + pb1_ref[...], 0.0)
    rp = _dot(t2, pw2_ref[...]) + pb2_ref[...]

    mx = mx_ref[...]
    mp = mp_ref[...]
    slx = jnp.sum(mx * (rx - x_ref[...]) ** 2)
    slp = jnp.sum(mp * (rp - posp_ref[...]) ** 2)

    cnt = cnt_ref[...]
    centro = csum_ref[...] * (1.0 / (cnt + 1e-6))
    u = _dot_t(w2r_ref[...], centro)
    v = _dot_t(b2r_ref[...], centro)
    m = jnp.mean(lx_ref[...], axis=1, keepdims=True)
    logits = (m * u + v) * 0.5
    mxl = jnp.max(logits, axis=1, keepdims=True)
    lse = jnp.log(jnp.sum(jnp.exp(logits - mxl), axis=1, keepdims=True)) + mxl
    iota = lax.broadcasted_iota(jnp.int32, (1, _NT), 1).astype(jnp.float32)
    onehot = (ctf_ref[...] == iota).astype(jnp.float32)
    picked = jnp.sum(onehot * logits, axis=1, keepdims=True)
    sct = jnp.sum(picked - lse)

    @pl.when(i == 0)
    def _init():
        slx_ref[...] = jnp.zeros_like(slx_ref)
        slp_ref[...] = jnp.zeros_like(slp_ref)
        smx_ref[...] = jnp.zeros_like(smx_ref)
        smp_ref[...] = jnp.zeros_like(smp_ref)
        sct_ref[...] = jnp.zeros_like(sct_ref)

    slx_ref[...] += slx.reshape(1, 1)
    slp_ref[...] += slp.reshape(1, 1)
    smx_ref[...] += jnp.sum(mx).reshape(1, 1)
    smp_ref[...] += jnp.sum(mp).reshape(1, 1)
    sct_ref[...] += sct.reshape(1, 1)


_row_spec = pl.BlockSpec((_BLK, _D), lambda i: (i, 0))
_col1_spec = pl.BlockSpec((_BLK, 1), lambda i: (i, 0))
_w_spec = pl.BlockSpec((_D, _D), lambda i: (0, 0))
_b_spec = pl.BlockSpec((1, _D), lambda i: (0, 0))
_full_f = jax.ShapeDtypeStruct((_N, _D), jnp.float32)
_sc_shape = jax.ShapeDtypeStruct((1, 1), jnp.float32)
_sc_spec = pl.BlockSpec((1, 1), lambda i: (0, 0))


def _mask_linear(x, mask, w, b):
    return pl.pallas_call(
        _mask_linear_body, grid=(_GRID,),
        in_specs=[_row_spec, _row_spec, _w_spec, _b_spec],
        out_specs=_row_spec, out_shape=_full_f,
    )(x, mask, w, b)


def _gin(h, part, w1, b1, w2, b2, relu_out):
    return pl.pallas_call(
        functools.partial(_gin_body, relu_out), grid=(_GRID,),
        in_specs=[_row_spec, _row_spec,
                  _w_spec, _b_spec, _w_spec, _b_spec],
        out_specs=_row_spec, out_shape=_full_f,
    )(h, part, w1, b1, w2, b2)


def _latent_stage(latent, mask_x, mask_pos, ctf, wc, bc):
    return pl.pallas_call(
        _latent_body, grid=(_GRID,),
        in_specs=[_row_spec, _row_spec, _col1_spec, _col1_spec, _w_spec, _b_spec],
        out_specs=[_row_spec,
                   pl.BlockSpec((_NT, _D), lambda i: (0, 0)),
                   pl.BlockSpec((_NT, 1), lambda i: (0, 0))],
        out_shape=[_full_f,
                   jax.ShapeDtypeStruct((_NT, _D), jnp.float32),
                   jax.ShapeDtypeStruct((_NT, 1), jnp.float32)],
    )(latent, mask_x, mask_pos, ctf, wc, bc)


def _decoder_stage(lx, part, x, posp, mask_x, mask_pos, ctf, csum, cnt,
                   xw1, xb1, xw2, xb2, pw1, pb1, pw2, pb2, w2r, b2r):
    return pl.pallas_call(
        _decoder_body, grid=(_GRID,),
        in_specs=[_row_spec, _row_spec, _row_spec, _row_spec, _row_spec,
                  _col1_spec, _col1_spec,
                  pl.BlockSpec((_NT, _D), lambda i: (0, 0)),
                  pl.BlockSpec((_NT, 1), lambda i: (0, 0)),
                  _w_spec, _b_spec, _w_spec, _b_spec,
                  _w_spec, _b_spec, _w_spec, _b_spec,
                  pl.BlockSpec((1, _D), lambda i: (0, 0)),
                  pl.BlockSpec((1, _D), lambda i: (0, 0))],
        out_specs=[_sc_spec] * 5,
        out_shape=[_sc_shape] * 5,
    )(lx, part, x, posp, mask_x, mask_pos, ctf, csum, cnt,
      xw1, xb1, xw2, xb2, pw1, pb1, pw2, pb2, w2r, b2r)


def _segsum_body(src_hbm, dst_hbm, h_hbm, zeros_hbm, out_hbm,
                 isrc, idst, buf, acc, *sems):
    c = lax.axis_index("c")
    s = lax.axis_index("s")
    base = s * _NCH
    gsem = sems[:_NBUF]
    zsem = sems[_NBUF:2 * _NBUF]
    ssem = sems[2 * _NBUF:2 * _NBUF + _IRING]
    dsem = sems[2 * _NBUF + _IRING:]

    def idx_cp(j, q):
        return (pltpu.make_async_copy(src_hbm.at[base + j], isrc.at[q],
                                      ssem[q]),
                pltpu.make_async_copy(dst_hbm.at[base + j], idst.at[q],
                                      dsem[q]))

    def gather_cp(b, q):
        return pltpu.make_async_copy(h_hbm.at[isrc.at[q]], buf.at[b], gsem[b])

    def scat_cp(b, q):
        return pltpu.make_async_copy(buf.at[b], acc.at[idst.at[q]], zsem[b])

    @pl.when(c == 1)
    def _run():
        for q in range(_IRING - 2):
            for cp in idx_cp(q, q):
                cp.start()
        pltpu.sync_copy(zeros_hbm.at[pl.ds(s * 640, 640)],
                        acc.at[pl.ds(s * 640, 640)])
        plsc.subcore_barrier()
        for b in range(_NBUF - 2):
            for cp in idx_cp(b, b):
                cp.wait()
            gather_cp(b, b).start()

        def step(t, carry):
            for u in range(2 * _NBUF):
                j = t * 2 * _NBUF + u
                b = u % _NBUF
                q = u % _IRING
                lb = (u + _NBUF - 2) % _NBUF
                lq = (u + _NBUF - 2) % _IRING
                gather_cp(b, q).wait()
                scat_cp(b, q).start(add=True)

                @pl.when(j + _NBUF - 2 < _NCH)
                def _():
                    @pl.when(j >= 2)
                    def _():
                        scat_cp(lb, (u + _IRING - 2) % _IRING).wait()

                    for cp in idx_cp(j + _NBUF - 2, lq):
                        cp.wait()
                    gather_cp(lb, lq).start()

                @pl.when(j + _IRING - 2 < _NCH)
                def _():
                    for cp in idx_cp(j + _IRING - 2, (u + _IRING - 2) % _IRING):
                        cp.start()

            return carry

        lax.fori_loop(0, _NCH // (2 * _NBUF), step, 0)
        for b in range(_NBUF):
            scat_cp(b, b).wait()
        plsc.subcore_barrier()
        pltpu.sync_copy(acc.at[pl.ds(s * 640, 640)],
                        out_hbm.at[pl.ds(s * 640, 640)])


def _segsum_sc(h, src_r, dst_r, zeros_acc):
    mesh = plsc.VectorSubcoreMesh(core_axis_name="c", subcore_axis_name="s")
    f = pl.kernel(
        _segsum_body, mesh=mesh,
        out_type=jax.ShapeDtypeStruct((_NACC, _D), jnp.float32),
        scratch_types=[
            pltpu.VMEM((_IRING, _CH), jnp.int32),
            pltpu.VMEM((_IRING, _CH), jnp.int32),
            pltpu.VMEM((_NBUF, _CH, _D), jnp.float32),
            pltpu.VMEM_SHARED((_NACC, _D), jnp.float32),
        ] + [pltpu.SemaphoreType.DMA] * (2 * _NBUF + 2 * _IRING),
    )
    return f(src_r, dst_r, h, zeros_acc)


def kernel(x, pos, edge_index, cell_type, batch, Wx, bx, encW1, encb1,
           encW2, encb2, Wc, bc, dxW1, dxb1, dxW2, dxb2, dpW1, dpb1,
           dpW2, dpb2, w2, b2, alpha):
    mkey = jax.random.key(123)
    k1, k2 = jax.random.split(mkey)
    mask_x = (jax.random.uniform(k1, x.shape) < 0.3).astype(jnp.float32)
    mask_pos = (jax.random.uniform(k2, (x.shape[0],)) < 0.3).astype(jnp.float32)
    mask_pos_c = mask_pos.reshape(_N, 1)

    src = edge_index[0].astype(jnp.int32)
    dst = edge_index[1].astype(jnp.int32)
    npad = _EPAD - src.shape[0]
    src_r = jnp.concatenate(
        [src, jnp.zeros((npad,), jnp.int32)]).reshape(_TOTCH, _CH)
    pad_dst = _N + (jnp.arange(npad, dtype=jnp.int32) % (_NACC - _N))
    dst_r = jnp.concatenate([dst, pad_dst]).reshape(_TOTCH, _CH)
    zeros_acc = jnp.zeros((_NACC, _D), jnp.float32)
    ctf = cell_type.astype(jnp.float32).reshape(_N, 1)
    posp = jnp.pad(pos, ((0, 0), (0, _D - 2)))
    dpW2p = jnp.pad(dpW2, ((0, 0), (0, _D - 2)))
    dpb2p = jnp.pad(dpb2, (0, _D - 2)).reshape(1, _D)
    bxr = bx.reshape(1, _D)
    bcr = bc.reshape(1, _D)
    w2r = w2.reshape(1, _D)
    b2r = b2.reshape(1, _D)

    h = _mask_linear(x, mask_x, Wx, bxr)
    for i in range(4):
        part = _segsum_sc(h, src_r, dst_r, zeros_acc)
        h = _gin(h, part, encW1[i], encb1[i].reshape(1, _D),
                 encW2[i], encb2[i].reshape(1, _D), relu_out=(i < 3))

    lx, csum, cnt = _latent_stage(h, mask_x, mask_pos_c, ctf, Wc, bcr)
    partd = _segsum_sc(lx, src_r, dst_r, zeros_acc)
    slx, slp, smx, smp, sct = _decoder_stage(
        lx, partd, x, posp, mask_x, mask_pos_c, ctf, csum, cnt,
        dxW1, dxb1.reshape(1, _D), dxW2, dxb2.reshape(1, _D),
        dpW1, dpb1.reshape(1, _D), dpW2p, dpb2p, w2r, b2r)

    loss_x = slx[0, 0] / (smx[0, 0] + 1e-6)
    loss_pos = slp[0, 0] / (2.0 * smp[0, 0] + 1e-6)
    contrastive = -sct[0, 0] / _N
    a = jax.nn.sigmoid(alpha)
    return a * contrastive + (1.0 - a) * (loss_x + loss_pos)


# speedup vs baseline: 1.2449x; 1.2449x over previous
import functools

import jax
import jax.numpy as jnp
from jax import lax
from jax.experimental import pallas as pl
from jax.experimental.pallas import tpu as pltpu
from jax.experimental.pallas import tpu_sc as plsc

_N = 10000
_D = 128
_NT = 16
_BLK = 400
_GRID = _N // _BLK

_NC = 2
_NS = 16
_CH = 64
_TOTCH = 5120
_NCH0 = 264
_NCH1 = _TOTCH // _NS - _NCH0
_NBUF = 4
_IRING = 8
_EPAD = _TOTCH * _CH
_NACC = _NS * 640


def _dot(a, b):
    return jnp.dot(a, b, preferred_element_type=jnp.float32)


def _dot_t(a, b):
    return lax.dot_general(a, b, (((1,), (1,)), ((), ())),
                           preferred_element_type=jnp.float32)


def _dot_c0(a, b):
    return lax.dot_general(a, b, (((0,), (0,)), ((), ())),
                           preferred_element_type=jnp.float32)


def _mask_linear_body(x_ref, mask_ref, w_ref, b_ref, o_ref):
    xm = x_ref[...] * (1.0 - mask_ref[...])
    o_ref[...] = _dot(xm, w_ref[...]) + b_ref[...]


def _gin_body(relu_out, h_ref, a0_ref, a1_ref, w1_ref, b1_ref, w2_ref, b2_ref,
              o_ref):
    a = h_ref[...] + a0_ref[0] + a1_ref[0]
    t = jnp.maximum(_dot(a, w1_ref[...]) + b1_ref[...], 0.0)
    o = _dot(t, w2_ref[...]) + b2_ref[...]
    if relu_out:
        o = jnp.maximum(o, 0.0)
    o_ref[...] = o


def _latent_body(lat_ref, mx_ref, mp_ref, ctf_ref, wc_ref, bc_ref,
                 lx_ref, csum_ref, cnt_ref):
    i = pl.program_id(0)
    lat = lat_ref[...]
    lpos = _dot(lat, wc_ref[...]) + bc_ref[...]
    lpos = jnp.where(mp_ref[...] > 0.0, 0.0, lpos)
    lx_ref[...] = jnp.where(mx_ref[...] > 0.0, 0.0, lat)
    iota = lax.broadcasted_iota(jnp.int32, (1, _NT), 1).astype(jnp.float32)
    onehot = (ctf_ref[...] == iota).astype(jnp.float32)
    csum_p = _dot_c0(onehot, lpos)
    cnt_p = _dot_c0(onehot, jnp.ones((_BLK, 1), jnp.float32))

    @pl.when(i == 0)
    def _init():
        csum_ref[...] = jnp.zeros_like(csum_ref)
        cnt_ref[...] = jnp.zeros_like(cnt_ref)

    csum_ref[...] += csum_p
    cnt_ref[...] += cnt_p


def _decoder_body(lx_ref, a0_ref, a1_ref, x_ref, posp_ref, mx_ref, mp_ref,
                  ctf_ref, csum_ref, cnt_ref,
                  xw1_ref, xb1_ref, xw2_ref, xb2_ref,
                  pw1_ref, pb1_ref, pw2_ref, pb2_ref,
                  w2r_ref, b2r_ref,
                  slx_ref, slp_ref, smx_ref, smp_ref, sct_ref):
    i = pl.program_id(0)
    a = lx_ref[...] + a0_ref[0] + a1_ref[0]
    t = jnp.maximum(_dot(a, xw1_ref[...]) + xb1_ref[...], 0.0)
    rx = _dot(t, xw2_ref[...]) + xb2_ref[...]
    t2 = jnp.maximum(_dot(a, pw1_ref[...]) + pb1_ref[...], 0.0)
    rp = _dot(t2, pw2_ref[...]) + pb2_ref[...]

    mx = mx_ref[...]
    mp = mp_ref[...]
    slx = jnp.sum(mx * (rx - x_ref[...]) ** 2)
    slp = jnp.sum(mp * (rp - posp_ref[...]) ** 2)

    cnt = cnt_ref[...]
    centro = csum_ref[...] * (1.0 / (cnt + 1e-6))
    u = _dot_t(w2r_ref[...], centro)
    v = _dot_t(b2r_ref[...], centro)
    m = jnp.mean(lx_ref[...], axis=1, keepdims=True)
    logits = (m * u + v) * 0.5
    mxl = jnp.max(logits, axis=1, keepdims=True)
    lse = jnp.log(jnp.sum(jnp.exp(logits - mxl), axis=1, keepdims=True)) + mxl
    iota = lax.broadcasted_iota(jnp.int32, (1, _NT), 1).astype(jnp.float32)
    onehot = (ctf_ref[...] == iota).astype(jnp.float32)
    picked = jnp.sum(onehot * logits, axis=1, keepdims=True)
    sct = jnp.sum(picked - lse)

    @pl.when(i == 0)
    def _init():
        slx_ref[...] = jnp.zeros_like(slx_ref)
        slp_ref[...] = jnp.zeros_like(slp_ref)
        smx_ref[...] = jnp.zeros_like(smx_ref)
        smp_ref[...] = jnp.zeros_like(smp_ref)
        sct_ref[...] = jnp.zeros_like(sct_ref)

    slx_ref[...] += slx.reshape(1, 1)
    slp_ref[...] += slp.reshape(1, 1)
    smx_ref[...] += jnp.sum(mx).reshape(1, 1)
    smp_ref[...] += jnp.sum(mp).reshape(1, 1)
    sct_ref[...] += sct.reshape(1, 1)


_row_spec = pl.BlockSpec((_BLK, _D), lambda i: (i, 0))
_col1_spec = pl.BlockSpec((_BLK, 1), lambda i: (i, 0))
_w_spec = pl.BlockSpec((_D, _D), lambda i: (0, 0))
_b_spec = pl.BlockSpec((1, _D), lambda i: (0, 0))
_full_f = jax.ShapeDtypeStruct((_N, _D), jnp.float32)
_sc_shape = jax.ShapeDtypeStruct((1, 1), jnp.float32)
_sc_spec = pl.BlockSpec((1, 1), lambda i: (0, 0))
_p0_spec = pl.BlockSpec((1, _BLK, _D), lambda i: (0, i, 0))
_p1_spec = pl.BlockSpec((1, _BLK, _D), lambda i: (1, i, 0))


def _mask_linear(x, mask, w, b):
    return pl.pallas_call(
        _mask_linear_body, grid=(_GRID,),
        in_specs=[_row_spec, _row_spec, _w_spec, _b_spec],
        out_specs=_row_spec, out_shape=_full_f,
    )(x, mask, w, b)


def _gin(h, parts, w1, b1, w2, b2, relu_out):
    return pl.pallas_call(
        functools.partial(_gin_body, relu_out), grid=(_GRID,),
        in_specs=[_row_spec, _p0_spec, _p1_spec,
                  _w_spec, _b_spec, _w_spec, _b_spec],
        out_specs=_row_spec, out_shape=_full_f,
    )(h, parts, parts, w1, b1, w2, b2)


def _latent_stage(latent, mask_x, mask_pos, ctf, wc, bc):
    return pl.pallas_call(
        _latent_body, grid=(_GRID,),
        in_specs=[_row_spec, _row_spec, _col1_spec, _col1_spec, _w_spec, _b_spec],
        out_specs=[_row_spec,
                   pl.BlockSpec((_NT, _D), lambda i: (0, 0)),
                   pl.BlockSpec((_NT, 1), lambda i: (0, 0))],
        out_shape=[_full_f,
                   jax.ShapeDtypeStruct((_NT, _D), jnp.float32),
                   jax.ShapeDtypeStruct((_NT, 1), jnp.float32)],
    )(latent, mask_x, mask_pos, ctf, wc, bc)


def _decoder_stage(lx, parts, x, posp, mask_x, mask_pos, ctf, csum, cnt,
                   xw1, xb1, xw2, xb2, pw1, pb1, pw2, pb2, w2r, b2r):
    return pl.pallas_call(
        _decoder_body, grid=(_GRID,),
        in_specs=[_row_spec, _p0_spec, _p1_spec, _row_spec, _row_spec,
                  _row_spec, _col1_spec, _col1_spec,
                  pl.BlockSpec((_NT, _D), lambda i: (0, 0)),
                  pl.BlockSpec((_NT, 1), lambda i: (0, 0)),
                  _w_spec, _b_spec, _w_spec, _b_spec,
                  _w_spec, _b_spec, _w_spec, _b_spec,
                  pl.BlockSpec((1, _D), lambda i: (0, 0)),
                  pl.BlockSpec((1, _D), lambda i: (0, 0))],
        out_specs=[_sc_spec] * 5,
        out_shape=[_sc_shape] * 5,
    )(lx, parts, parts, x, posp, mask_x, mask_pos, ctf, csum, cnt,
      xw1, xb1, xw2, xb2, pw1, pb1, pw2, pb2, w2r, b2r)


def _segsum_body(src_hbm, dst_hbm, h_hbm, zeros_hbm, out_hbm,
                 isrc, idst, buf, acc, *sems):
    c = lax.axis_index("c")
    s = lax.axis_index("s")
    nch = jnp.where(c == 0, _NCH0, _NCH1)
    base = jnp.where(c == 0, s * _NCH0, _NS * _NCH0 + s * _NCH1)
    gsem = sems[:_NBUF]
    ssem = sems[_NBUF:_NBUF + _IRING]
    dsem = sems[_NBUF + _IRING:]

    def idx_cp(j, q):
        return (pltpu.make_async_copy(src_hbm.at[base + j], isrc.at[q],
                                      ssem[q]),
                pltpu.make_async_copy(dst_hbm.at[base + j], idst.at[q],
                                      dsem[q]))

    def gather_cp(b, q):
        return pltpu.make_async_copy(h_hbm.at[isrc.at[q]], buf.at[b], gsem[b])

    for q in range(_IRING):
        for cp in idx_cp(q, q):
            cp.start()
    pltpu.sync_copy(zeros_hbm.at[pl.ds(s * 640, 640)],
                    acc.at[pl.ds(s * 640, 640)])
    plsc.subcore_barrier()
    for b in range(_NBUF - 1):
        for cp in idx_cp(b, b):
            cp.wait()
        gather_cp(b, b).start()

    def step(t, carry):
        for q in range(_IRING):
            j = t * _IRING + q
            b = q % _NBUF
            nb = (q + _NBUF - 1) % _NBUF
            nq = (q + _NBUF - 1) % _IRING
            gather_cp(b, q).wait()

            @pl.when(j + _NBUF - 1 < nch)
            def _():
                for cp in idx_cp(j + _NBUF - 1, nq):
                    cp.wait()
                gather_cp(nb, nq).start()

            pltpu.sync_copy(buf.at[b], acc.at[idst.at[q]], add=True)

            @pl.when(j + _IRING < nch)
            def _():
                for cp in idx_cp(j + _IRING, q):
                    cp.start()

        return carry

    lax.fori_loop(0, nch // _IRING, step, 0)
    plsc.subcore_barrier()
    pltpu.sync_copy(acc.at[pl.ds(s * 640, 640)],
                    out_hbm.at[c].at[pl.ds(s * 640, 640)])


def _segsum_sc(h, src_r, dst_r, zeros_acc):
    mesh = plsc.VectorSubcoreMesh(core_axis_name="c", subcore_axis_name="s")
    f = pl.kernel(
        _segsum_body, mesh=mesh,
        out_type=jax.ShapeDtypeStruct((_NC, _NACC, _D), jnp.float32),
        scratch_types=[
            pltpu.VMEM((_IRING, _CH), jnp.int32),
            pltpu.VMEM((_IRING, _CH), jnp.int32),
            pltpu.VMEM((_NBUF, _CH, _D), jnp.float32),
            pltpu.VMEM_SHARED((_NACC, _D), jnp.float32),
        ] + [pltpu.SemaphoreType.DMA] * (_NBUF + 2 * _IRING),
    )
    return f(src_r, dst_r, h, zeros_acc)


def kernel(x, pos, edge_index, cell_type, batch, Wx, bx, encW1, encb1,
           encW2, encb2, Wc, bc, dxW1, dxb1, dxW2, dxb2, dpW1, dpb1,
           dpW2, dpb2, w2, b2, alpha):
    mkey = jax.random.key(123)
    k1, k2 = jax.random.split(mkey)
    mask_x = (jax.random.uniform(k1, x.shape) < 0.3).astype(jnp.float32)
    mask_pos = (jax.random.uniform(k2, (x.shape[0],)) < 0.3).astype(jnp.float32)
    mask_pos_c = mask_pos.reshape(_N, 1)

    src = edge_index[0].astype(jnp.int32)
    dst = edge_index[1].astype(jnp.int32)
    npad = _EPAD - src.shape[0]
    src_r = jnp.concatenate(
        [src, jnp.zeros((npad,), jnp.int32)]).reshape(_TOTCH, _CH)
    pad_dst = _N + (jnp.arange(npad, dtype=jnp.int32) % (_NACC - _N))
    dst_r = jnp.concatenate([dst, pad_dst]).reshape(_TOTCH, _CH)
    zeros_acc = jnp.zeros((_NACC, _D), jnp.float32)
    ctf = cell_type.astype(jnp.float32).reshape(_N, 1)
    posp = jnp.pad(pos, ((0, 0), (0, _D - 2)))
    dpW2p = jnp.pad(dpW2, ((0, 0), (0, _D - 2)))
    dpb2p = jnp.pad(dpb2, (0, _D - 2)).reshape(1, _D)
    bxr = bx.reshape(1, _D)
    bcr = bc.reshape(1, _D)
    w2r = w2.reshape(1, _D)
    b2r = b2.reshape(1, _D)

    h = _mask_linear(x, mask_x, Wx, bxr)
    for i in range(4):
        parts = _segsum_sc(h, src_r, dst_r, zeros_acc)
        h = _gin(h, parts, encW1[i], encb1[i].reshape(1, _D),
                 encW2[i], encb2[i].reshape(1, _D), relu_out=(i < 3))

    lx, csum, cnt = _latent_stage(h, mask_x, mask_pos_c, ctf, Wc, bcr)
    partsd = _segsum_sc(lx, src_r, dst_r, zeros_acc)
    slx, slp, smx, smp, sct = _decoder_stage(
        lx, partsd, x, posp, mask_x, mask_pos_c, ctf, csum, cnt,
        dxW1, dxb1.reshape(1, _D), dxW2, dxb2.reshape(1, _D),
        dpW1, dpb1.reshape(1, _D), dpW2p, dpb2p, w2r, b2r)

    loss_x = slx[0, 0] / (smx[0, 0] + 1e-6)
    loss_pos = slp[0, 0] / (2.0 * smp[0, 0] + 1e-6)
    contrastive = -sct[0, 0] / _N
    a = jax.nn.sigmoid(alpha)
    return a * contrastive + (1.0 - a) * (loss_x + loss_pos)
